# baseline (speedup 1.0000x reference)
import functools

import jax
import jax.numpy as jnp
from jax import lax
from jax.experimental import pallas as pl
from jax.experimental.pallas import tpu as pltpu
from jax.experimental.pallas import tpu_sc as plsc

NC = 2
NS = 16
CHUNK = 128
IB = 16


def _seg_sum_sc(features, src_idx, dst_idx, zeros_rows, ones_rows,
                n_pad, cpt0, cpt1, ct0, ct1):
    feat = features.shape[1]
    rpt = n_pad // NS

    @functools.partial(
        pl.kernel,
        out_type=[
            jax.ShapeDtypeStruct((2 * n_pad, feat), jnp.float32),
            jax.ShapeDtypeStruct((2 * n_pad, feat), jnp.float32),
        ],
        mesh=plsc.VectorSubcoreMesh(core_axis_name="c", subcore_axis_name="s"),
        scratch_types=[
            pltpu.VMEM_SHARED((n_pad, feat), jnp.float32),
            pltpu.VMEM((IB, CHUNK), jnp.int32),
            pltpu.VMEM((IB, CHUNK), jnp.int32),
            pltpu.VMEM((CHUNK, feat), jnp.float32),
            pltpu.VMEM((CHUNK, feat), jnp.float32),
            pltpu.SemaphoreType.DMA,
            pltpu.SemaphoreType.DMA,
            pltpu.SemaphoreType.DMA,
            pltpu.SemaphoreType.DMA,
        ],
    )
    def k(feat_hbm, src_hbm, dst_hbm, zrows_hbm, ones_hbm,
          seg_hbm, cnt_hbm,
          acc_sh, src_v, dst_v, gbuf, gbuf2,
          sem_g0, sem_g1, sem_s0, sem_s1):
        c = lax.axis_index("c")
        s = lax.axis_index("s")
        bufs = (gbuf, gbuf2)
        sems_g = (sem_g0, sem_g1)
        sems_s = (sem_s0, sem_s1)

        def zero_acc():
            pltpu.sync_copy(zrows_hbm, gbuf)
            for t in range(rpt // CHUNK):
                pltpu.sync_copy(
                    gbuf, acc_sh.at[pl.ds(s * rpt + t * CHUNK, CHUNK)])

        def write_acc(out_hbm):
            for t in range(rpt // CHUNK):
                pltpu.sync_copy(
                    acc_sh.at[pl.ds(s * rpt + t * CHUNK, CHUNK)], gbuf)
                pltpu.sync_copy(
                    gbuf,
                    out_hbm.at[pl.ds(c * n_pad + s * rpt + t * CHUNK,
                                     CHUNK)])

        def seg_pass(cpt_c, base):
            @pl.loop(0, cpt_c // IB)
            def _(b):
                off = base + b * IB
                pltpu.sync_copy(dst_hbm.at[pl.ds(off, IB)], dst_v)
                pltpu.sync_copy(src_hbm.at[pl.ds(off, IB)], src_v)
                pend_g = [None, None]
                pend_s = [None, None]
                pend_g[0] = pltpu.async_copy(
                    feat_hbm.at[src_v.at[0]], bufs[0], sems_g[0])
                for j in range(IB):
                    p = j % 2
                    pend_g[p].wait()
                    if j + 1 < IB:
                        q = (j + 1) % 2
                        if pend_s[q] is not None:
                            pend_s[q].wait()
                        pend_g[q] = pltpu.async_copy(
                            feat_hbm.at[src_v.at[j + 1]], bufs[q],
                            sems_g[q])
                    pend_s[p] = pltpu.async_copy(
                        bufs[p], acc_sh.at[dst_v.at[j]], sems_s[p],
                        add=True)
                pend_s[0].wait()
                pend_s[1].wait()

        def cnt_pass(cpt_c, base):
            @pl.loop(0, cpt_c // IB)
            def _(b):
                off = base + b * IB
                pltpu.sync_copy(dst_hbm.at[pl.ds(off, IB)], dst_v)
                pend = []
                for j in range(IB):
                    pend.append(pltpu.async_copy(
                        gbuf, acc_sh.at[dst_v.at[j]], sem_s0, add=True))
                for p in pend:
                    p.wait()

        zero_acc()
        plsc.subcore_barrier()

        @pl.when(c == 0)
        def _():
            seg_pass(cpt0, s * cpt0)
        @pl.when(c == 1)
        def _():
            seg_pass(cpt1, NS * cpt0 + s * cpt1)

        plsc.subcore_barrier()
        write_acc(seg_hbm)
        plsc.subcore_barrier()

        zero_acc()
        pltpu.sync_copy(ones_hbm, gbuf)
        plsc.subcore_barrier()

        @pl.when(c == 0)
        def _():
            cnt_pass(ct0, s * ct0)
        @pl.when(c == 1)
        def _():
            cnt_pass(ct1, NS * ct0 + s * ct1)

        plsc.subcore_barrier()
        write_acc(cnt_hbm)

    return k(features, src_idx, dst_idx, zeros_rows, ones_rows)


def _combine_tc(features, seg, cnt, weight, n_nodes, n_pad, blk):
    out_feat = weight.shape[1]

    def body(f_ref, s0_ref, s1_ref, c0_ref, c1_ref, w_ref, o_ref):
        w = w_ref[...]
        nodes = jnp.dot(f_ref[...], w, preferred_element_type=jnp.float32)
        cnt_blk = (c0_ref[...] + c1_ref[...])[:, :1]
        inv = jnp.where(cnt_blk > 0, 1.0 / cnt_blk, 0.0)
        ssum = s0_ref[...] + s1_ref[...]
        msg = jnp.dot(ssum * inv, w, preferred_element_type=jnp.float32)
        o_ref[...] = jnp.concatenate([nodes, msg], axis=-1)

    grid = n_nodes // blk
    rowspec = pl.BlockSpec((blk, features.shape[1]), lambda i: (i, 0))
    return pl.pallas_call(
        body,
        grid=(grid,),
        in_specs=[rowspec] * 5 + [pl.BlockSpec(weight.shape, lambda i: (0, 0))],
        out_specs=pl.BlockSpec((blk, 2 * out_feat), lambda i: (i, 0)),
        out_shape=jax.ShapeDtypeStruct((n_nodes, 2 * out_feat), jnp.float32),
    )(features, seg[:n_nodes], seg[n_pad:n_pad + n_nodes],
      cnt[:n_nodes], cnt[n_pad:n_pad + n_nodes], weight)


def kernel(features, edges, static_adjacency_matrix, adjacency_matrix, weight):
    del static_adjacency_matrix, adjacency_matrix
    n_nodes, in_feat = features.shape
    e = edges.shape[1]

    n_pad = ((n_nodes + NS * CHUNK) // (NS * CHUNK)) * NS * CHUNK
    cpt = (e + NS * CHUNK - 1) // (NS * CHUNK)
    cpt = ((cpt + NC * IB - 1) // (NC * IB)) * NC * IB
    cpt0 = max(IB, ((cpt * 6 // 10) // IB) * IB)
    cpt1 = cpt - cpt0
    ct0 = max(IB, ((cpt * 10 // 10) // IB) * IB)
    ct1 = cpt - ct0
    e_pad = NS * CHUNK * cpt

    src = jnp.concatenate(
        [edges[1], jnp.zeros((e_pad - e,), jnp.int32)]).reshape(
        NS * cpt, CHUNK)
    dst = jnp.concatenate(
        [edges[0], jnp.full((e_pad - e,), n_nodes, jnp.int32)]).reshape(
        NS * cpt, CHUNK)

    zeros_rows = jnp.zeros((CHUNK, in_feat), jnp.float32)
    ones_rows = jnp.ones((CHUNK, in_feat), jnp.float32)

    seg, cnt = _seg_sum_sc(features, src, dst, zeros_rows, ones_rows,
                           n_pad, cpt0, cpt1, ct0, ct1)

    blk = 1000 if n_nodes % 1000 == 0 else 8
    return _combine_tc(features, seg, cnt, weight, n_nodes, n_pad, blk)


# speedup vs baseline: 1.0221x; 1.0221x over previous
import functools

import jax
import jax.numpy as jnp
from jax import lax
from jax.experimental import pallas as pl
from jax.experimental.pallas import tpu as pltpu
from jax.experimental.pallas import tpu_sc as plsc

NC = 2
NS = 16
CHUNK = 128
IB = 16


def _seg_sum_sc(features, src_idx, dst_idx, zeros_rows, ones_rows,
                n_pad, cpt0, cpt1, ct0, ct1):
    feat = features.shape[1]
    rpt = n_pad // NS

    @functools.partial(
        pl.kernel,
        out_type=[
            jax.ShapeDtypeStruct((2 * n_pad, feat), jnp.float32),
            jax.ShapeDtypeStruct((2 * n_pad, feat), jnp.float32),
        ],
        mesh=plsc.VectorSubcoreMesh(core_axis_name="c", subcore_axis_name="s"),
        scratch_types=[
            pltpu.VMEM_SHARED((n_pad, feat), jnp.float32),
            pltpu.VMEM((IB, CHUNK), jnp.int32),
            pltpu.VMEM((IB, CHUNK), jnp.int32),
            pltpu.VMEM((CHUNK, feat), jnp.float32),
            pltpu.VMEM((CHUNK, feat), jnp.float32),
            pltpu.SemaphoreType.DMA,
            pltpu.SemaphoreType.DMA,
            pltpu.SemaphoreType.DMA,
            pltpu.SemaphoreType.DMA,
        ],
    )
    def k(feat_hbm, src_hbm, dst_hbm, zrows_hbm, ones_hbm,
          seg_hbm, cnt_hbm,
          acc_sh, src_v, dst_v, gbuf, gbuf2,
          sem_g0, sem_g1, sem_s0, sem_s1):
        c = lax.axis_index("c")
        s = lax.axis_index("s")
        bufs = (gbuf, gbuf2)
        sems_g = (sem_g0, sem_g1)
        sems_s = (sem_s0, sem_s1)

        def zero_acc():
            pltpu.sync_copy(zrows_hbm, gbuf)
            for t in range(rpt // CHUNK):
                pltpu.sync_copy(
                    gbuf, acc_sh.at[pl.ds(s * rpt + t * CHUNK, CHUNK)])

        def write_acc(out_hbm):
            for t in range(rpt // CHUNK):
                pltpu.sync_copy(
                    acc_sh.at[pl.ds(s * rpt + t * CHUNK, CHUNK)], gbuf)
                pltpu.sync_copy(
                    gbuf,
                    out_hbm.at[pl.ds(c * n_pad + s * rpt + t * CHUNK,
                                     CHUNK)])

        def seg_pass(cpt_c, base):
            @pl.loop(0, cpt_c // IB)
            def _(b):
                off = base + b * IB
                pltpu.sync_copy(dst_hbm.at[pl.ds(off, IB)], dst_v)
                pltpu.sync_copy(src_hbm.at[pl.ds(off, IB)], src_v)
                pend_g = [None, None]
                pend_s = [None, None]
                pend_g[0] = pltpu.async_copy(
                    feat_hbm.at[src_v.at[0]], bufs[0], sems_g[0])
                for j in range(IB):
                    p = j % 2
                    pend_g[p].wait()
                    if j + 1 < IB:
                        q = (j + 1) % 2
                        if pend_s[q] is not None:
                            pend_s[q].wait()
                        pend_g[q] = pltpu.async_copy(
                            feat_hbm.at[src_v.at[j + 1]], bufs[q],
                            sems_g[q])
                    pend_s[p] = pltpu.async_copy(
                        bufs[p], acc_sh.at[dst_v.at[j]], sems_s[p],
                        add=True)
                pend_s[0].wait()
                pend_s[1].wait()

        def cnt_pass(cpt_c, base):
            @pl.loop(0, cpt_c // IB)
            def _(b):
                off = base + b * IB
                pltpu.sync_copy(dst_hbm.at[pl.ds(off, IB)], dst_v)
                pend = []
                for j in range(IB):
                    pend.append(pltpu.async_copy(
                        gbuf, acc_sh.at[dst_v.at[j]], sem_s0, add=True))
                for p in pend:
                    p.wait()

        zero_acc()
        plsc.subcore_barrier()

        @pl.when(c == 0)
        def _():
            seg_pass(cpt0, s * cpt0)
        @pl.when(c == 1)
        def _():
            seg_pass(cpt1, NS * cpt0 + s * cpt1)

        plsc.subcore_barrier()
        write_acc(seg_hbm)
        plsc.subcore_barrier()

        zero_acc()
        pltpu.sync_copy(ones_hbm, gbuf)
        plsc.subcore_barrier()

        @pl.when(c == 0)
        def _():
            cnt_pass(ct0, s * ct0)
        @pl.when(c == 1)
        def _():
            cnt_pass(ct1, NS * ct0 + s * ct1)

        plsc.subcore_barrier()
        write_acc(cnt_hbm)

    return k(features, src_idx, dst_idx, zeros_rows, ones_rows)


def _combine_tc(features, seg, cnt, weight, n_nodes, n_pad, blk):
    out_feat = weight.shape[1]

    def body(f_ref, s0_ref, s1_ref, c0_ref, c1_ref, w_ref, o_ref):
        w = w_ref[...]
        nodes = jnp.dot(f_ref[...], w, preferred_element_type=jnp.float32)
        cnt_blk = (c0_ref[...] + c1_ref[...])[:, :1]
        inv = jnp.where(cnt_blk > 0, 1.0 / cnt_blk, 0.0)
        ssum = s0_ref[...] + s1_ref[...]
        msg = jnp.dot(ssum * inv, w, preferred_element_type=jnp.float32)
        o_ref[...] = jnp.concatenate([nodes, msg], axis=-1)

    grid = n_nodes // blk
    rowspec = pl.BlockSpec((blk, features.shape[1]), lambda i: (i, 0))
    return pl.pallas_call(
        body,
        grid=(grid,),
        in_specs=[rowspec] * 5 + [pl.BlockSpec(weight.shape, lambda i: (0, 0))],
        out_specs=pl.BlockSpec((blk, 2 * out_feat), lambda i: (i, 0)),
        out_shape=jax.ShapeDtypeStruct((n_nodes, 2 * out_feat), jnp.float32),
    )(features, seg[:n_nodes], seg[n_pad:n_pad + n_nodes],
      cnt[:n_nodes], cnt[n_pad:n_pad + n_nodes], weight)


def kernel(features, edges, static_adjacency_matrix, adjacency_matrix, weight):
    del static_adjacency_matrix, adjacency_matrix
    n_nodes, in_feat = features.shape
    e = edges.shape[1]

    n_pad = ((n_nodes + NS * CHUNK) // (NS * CHUNK)) * NS * CHUNK
    cpt = (e + NS * CHUNK - 1) // (NS * CHUNK)
    cpt = ((cpt + NC * IB - 1) // (NC * IB)) * NC * IB
    cpt0 = max(IB, ((cpt * 75 // 100) // IB) * IB)
    cpt1 = cpt - cpt0
    ct0 = max(IB, ((cpt * 10 // 10) // IB) * IB)
    ct1 = cpt - ct0
    e_pad = NS * CHUNK * cpt

    src = jnp.concatenate(
        [edges[1], jnp.zeros((e_pad - e,), jnp.int32)]).reshape(
        NS * cpt, CHUNK)
    dst = jnp.concatenate(
        [edges[0], jnp.full((e_pad - e,), n_nodes, jnp.int32)]).reshape(
        NS * cpt, CHUNK)

    zeros_rows = jnp.zeros((CHUNK, in_feat), jnp.float32)
    ones_rows = jnp.ones((CHUNK, in_feat), jnp.float32)

    seg, cnt = _seg_sum_sc(features, src, dst, zeros_rows, ones_rows,
                           n_pad, cpt0, cpt1, ct0, ct1)

    blk = 1000 if n_nodes % 1000 == 0 else 8
    return _combine_tc(features, seg, cnt, weight, n_nodes, n_pad, blk)


# speedup vs baseline: 1.0645x; 1.0415x over previous
import functools

import jax
import jax.numpy as jnp
from jax import lax
from jax.experimental import pallas as pl
from jax.experimental.pallas import tpu as pltpu
from jax.experimental.pallas import tpu_sc as plsc

NC = 2
NS = 16
CHUNK = 128
IB = 16


def _seg_sum_sc(features, src_idx, dst_idx, zeros_rows, ones_rows,
                n_pad, cpt0, cpt1):
    feat = features.shape[1]
    rpt = n_pad // NS

    @functools.partial(
        pl.kernel,
        out_type=[
            jax.ShapeDtypeStruct((2 * n_pad, feat), jnp.float32),
            jax.ShapeDtypeStruct((n_pad, feat), jnp.float32),
        ],
        mesh=plsc.VectorSubcoreMesh(core_axis_name="c", subcore_axis_name="s"),
        scratch_types=[
            pltpu.VMEM_SHARED((n_pad, feat), jnp.float32),
            pltpu.VMEM((IB, CHUNK), jnp.int32),
            pltpu.VMEM((IB, CHUNK), jnp.int32),
            pltpu.VMEM((CHUNK, feat), jnp.float32),
            pltpu.VMEM((CHUNK, feat), jnp.float32),
            pltpu.SemaphoreType.DMA,
            pltpu.SemaphoreType.DMA,
            pltpu.SemaphoreType.DMA,
            pltpu.SemaphoreType.DMA,
        ],
    )
    def k(feat_hbm, src_hbm, dst_hbm, zrows_hbm, ones_hbm,
          seg_hbm, cnt_hbm,
          acc_sh, src_v, dst_v, gbuf, gbuf2,
          sem_g0, sem_g1, sem_s0, sem_s1):
        c = lax.axis_index("c")
        s = lax.axis_index("s")
        bufs = (gbuf, gbuf2)
        sems_g = (sem_g0, sem_g1)
        sems_s = (sem_s0, sem_s1)

        def zero_acc():
            pltpu.sync_copy(zrows_hbm, gbuf)
            for t in range(rpt // CHUNK):
                pltpu.sync_copy(
                    gbuf, acc_sh.at[pl.ds(s * rpt + t * CHUNK, CHUNK)])

        def write_acc(out_hbm, core_offset):
            for t in range(rpt // CHUNK):
                pltpu.sync_copy(
                    acc_sh.at[pl.ds(s * rpt + t * CHUNK, CHUNK)], gbuf)
                pltpu.sync_copy(
                    gbuf,
                    out_hbm.at[pl.ds(core_offset + s * rpt + t * CHUNK,
                                     CHUNK)])

        def seg_pass(cpt_c, base):
            @pl.loop(0, cpt_c // IB)
            def _(b):
                off = base + b * IB
                pltpu.sync_copy(dst_hbm.at[pl.ds(off, IB)], dst_v)
                pltpu.sync_copy(src_hbm.at[pl.ds(off, IB)], src_v)
                pend_g = [None, None]
                pend_s = [None, None]
                pend_g[0] = pltpu.async_copy(
                    feat_hbm.at[src_v.at[0]], bufs[0], sems_g[0])
                for j in range(IB):
                    p = j % 2
                    pend_g[p].wait()
                    if j + 1 < IB:
                        q = (j + 1) % 2
                        if pend_s[q] is not None:
                            pend_s[q].wait()
                        pend_g[q] = pltpu.async_copy(
                            feat_hbm.at[src_v.at[j + 1]], bufs[q],
                            sems_g[q])
                    pend_s[p] = pltpu.async_copy(
                        bufs[p], acc_sh.at[dst_v.at[j]], sems_s[p],
                        add=True)
                pend_s[0].wait()
                pend_s[1].wait()

        def cnt_pass(cpt_c, base):
            @pl.loop(0, cpt_c // IB)
            def _(b):
                off = base + b * IB
                pltpu.sync_copy(dst_hbm.at[pl.ds(off, IB)], dst_v)
                pend = []
                for j in range(IB):
                    pend.append(pltpu.async_copy(
                        gbuf, acc_sh.at[dst_v.at[j]], sem_s0, add=True))
                for p in pend:
                    p.wait()

        zero_acc()
        plsc.subcore_barrier()

        @pl.when(c == 0)
        def _():
            seg_pass(cpt0, s * cpt0)
        @pl.when(c == 1)
        def _():
            seg_pass(cpt1, NS * cpt0 + s * cpt1)

        plsc.subcore_barrier()
        write_acc(seg_hbm, c * n_pad)
        plsc.subcore_barrier()

        @pl.when(c == 0)
        def _():
            zero_acc()
            pltpu.sync_copy(ones_hbm, gbuf)
            plsc.subcore_barrier()
            cnt_pass(cpt0 + cpt1, s * (cpt0 + cpt1))
            plsc.subcore_barrier()
            write_acc(cnt_hbm, 0)

    return k(features, src_idx, dst_idx, zeros_rows, ones_rows)


def _combine_tc(features, seg, cnt, weight, n_nodes, n_pad, blk):
    out_feat = weight.shape[1]

    def body(f_ref, s0_ref, s1_ref, c_ref, w_ref, o_ref):
        w = w_ref[...]
        nodes = jnp.dot(f_ref[...], w, preferred_element_type=jnp.float32)
        cnt_blk = c_ref[...][:, :1]
        inv = jnp.where(cnt_blk > 0, 1.0 / cnt_blk, 0.0)
        ssum = s0_ref[...] + s1_ref[...]
        msg = jnp.dot(ssum * inv, w, preferred_element_type=jnp.float32)
        o_ref[...] = jnp.concatenate([nodes, msg], axis=-1)

    grid = n_nodes // blk
    rowspec = pl.BlockSpec((blk, features.shape[1]), lambda i: (i, 0))
    return pl.pallas_call(
        body,
        grid=(grid,),
        in_specs=[rowspec] * 4 + [pl.BlockSpec(weight.shape, lambda i: (0, 0))],
        out_specs=pl.BlockSpec((blk, 2 * out_feat), lambda i: (i, 0)),
        out_shape=jax.ShapeDtypeStruct((n_nodes, 2 * out_feat), jnp.float32),
    )(features, seg[:n_nodes], seg[n_pad:n_pad + n_nodes],
      cnt[:n_nodes], weight)


def kernel(features, edges, static_adjacency_matrix, adjacency_matrix, weight):
    del static_adjacency_matrix, adjacency_matrix
    n_nodes, in_feat = features.shape
    e = edges.shape[1]

    n_pad = ((n_nodes + NS * CHUNK) // (NS * CHUNK)) * NS * CHUNK
    cpt = (e + NS * CHUNK - 1) // (NS * CHUNK)
    cpt = ((cpt + NC * IB - 1) // (NC * IB)) * NC * IB
    cpt0 = max(IB, ((cpt * 75 // 100) // IB) * IB)
    cpt1 = cpt - cpt0
    e_pad = NS * CHUNK * cpt

    src = jnp.concatenate(
        [edges[1], jnp.zeros((e_pad - e,), jnp.int32)]).reshape(
        NS * cpt, CHUNK)
    dst = jnp.concatenate(
        [edges[0], jnp.full((e_pad - e,), n_nodes, jnp.int32)]).reshape(
        NS * cpt, CHUNK)

    zeros_rows = jnp.zeros((CHUNK, in_feat), jnp.float32)
    ones_rows = jnp.ones((CHUNK, in_feat), jnp.float32)

    seg, cnt = _seg_sum_sc(features, src, dst, zeros_rows, ones_rows,
                           n_pad, cpt0, cpt1)

    blk = 1000 if n_nodes % 1000 == 0 else 8
    return _combine_tc(features, seg, cnt, weight, n_nodes, n_pad, blk)
